# Initial kernel scaffold; baseline (speedup 1.0000x reference)
import jax
import jax.numpy as jnp
from jax.experimental import pallas as pl


def kernel(x, edge_index, W11l, W11r, b11, g11, bt11, W12l, W12r, b12, g12, bt12, W21l, W21r, b21, g21, bt21, W22l, W22r, b22, g22, bt22):
    raise NotImplementedError("write your pallas kernel here")


# speedup vs baseline: 7.8087x; 7.8087x over previous
import functools

import jax
import jax.numpy as jnp
import numpy as np
from jax import lax
from jax.experimental import pallas as pl
from jax.experimental.pallas import tpu as pltpu
from jax.experimental.pallas import tpu_sc as plsc

N, E, C = 10000, 320000, 128
NC, NS = 2, 16
N_PAD = 10240
CHUNK = 128

PT1 = 10112
EP1 = PT1 * NS
PT2 = 20096
EP2 = PT2 * NS

_ma = np.zeros((C, C), np.float32)
for _c in range(C):
    for _k in range(_c - 12, _c + 13):
        _ma[min(max(_k, 0), C - 1), _c] += 1.0 / 25.0
_MA = _ma


def _pad_edges(src, dst, per_core, n_cores_split):
    srcs, dsts = [], []
    e_half = E // n_cores_split
    for c in range(n_cores_split):
        s = src[c * e_half:(c + 1) * e_half]
        d = dst[c * e_half:(c + 1) * e_half]
        npad = per_core - e_half
        j = jnp.arange(npad, dtype=jnp.int32)
        srcs.append(jnp.concatenate([s, j % 997]))
        dsts.append(jnp.concatenate([d, N + (j % (N_PAD - N))]))
    return jnp.stack(srcs), jnp.stack(dsts)


def _make_sc_agg(t_rows, pt, with_deg):
    n_chunks = pt // CHUNK
    rows_per_tile = N_PAD // NS
    mesh = plsc.VectorSubcoreMesh(core_axis_name="c", subcore_axis_name="s")
    out_type = [jax.ShapeDtypeStruct((NC, N_PAD, C), jnp.float32)]
    if with_deg:
        out_type.append(jax.ShapeDtypeStruct((NC, N_PAD), jnp.float32))
    scratch = [
        pltpu.VMEM_SHARED((N_PAD, C), jnp.float32),
        pltpu.VMEM((CHUNK,), jnp.int32),
        pltpu.VMEM((CHUNK,), jnp.int32),
        pltpu.VMEM((CHUNK, C), jnp.float32),
        pltpu.SemaphoreType.DMA,
    ]
    if with_deg:
        scratch.insert(1, pltpu.VMEM_SHARED((N_PAD,), jnp.float32))
        scratch.append(pltpu.VMEM((CHUNK,), jnp.float32))

    @functools.partial(pl.kernel, out_type=out_type, mesh=mesh,
                       scratch_types=scratch, name="sc_edge_agg")
    def k(table_h, srcs_h, dsts_h, zeros_h, *refs):
        if with_deg:
            (zeros1_h, acc_out, deg_out, acc_sh, deg_sh, sidx_v, didx_v,
             rows_v, sem, ones_v) = refs
        else:
            acc_out, acc_sh, sidx_v, didx_v, rows_v, sem = refs
        c = lax.axis_index("c")
        s = lax.axis_index("s")

        zslc = pl.ds(s * rows_per_tile, rows_per_tile)
        pltpu.sync_copy(zeros_h.at[zslc], acc_sh.at[zslc])
        if with_deg:
            pltpu.sync_copy(zeros1_h.at[zslc], deg_sh.at[zslc])
            for i in range(CHUNK // 16):
                ones_v[pl.ds(i * 16, 16)] = jnp.full((16,), 1.0, jnp.float32)
        plsc.subcore_barrier()

        base = s * pt

        def body(j, _):
            off = base + j * CHUNK
            pltpu.sync_copy(srcs_h.at[c, pl.ds(off, CHUNK)], sidx_v)
            pltpu.sync_copy(dsts_h.at[c, pl.ds(off, CHUNK)], didx_v)
            pltpu.async_copy(table_h.at[sidx_v], rows_v, sem).wait()
            pltpu.sync_copy(rows_v, acc_sh.at[didx_v], add=True)
            if with_deg:
                pltpu.sync_copy(ones_v, deg_sh.at[didx_v], add=True)
            return ()

        lax.fori_loop(0, n_chunks, body, ())
        plsc.subcore_barrier()

        pltpu.sync_copy(acc_sh.at[zslc], acc_out.at[c, zslc])
        if with_deg:
            pltpu.sync_copy(deg_sh.at[zslc], deg_out.at[c, zslc])

    return k


_sc_agg1 = _make_sc_agg(N, PT1, with_deg=True)
_sc_agg2 = _make_sc_agg(2 * N, PT2, with_deg=False)


_HI = lax.Precision.HIGHEST
BR = 2000
NB = N // BR

def _row_spec(shape):
    return pl.BlockSpec(shape, lambda i: (i,) + (0,) * (len(shape) - 1))


def _stk_spec(shape):
    return pl.BlockSpec(shape, lambda i: (0, i, 0))


def _fix_spec(shape):
    return pl.BlockSpec(shape, lambda i: (0,) * len(shape))


def _stats_accum(stats_ref, i, a, b):
    @pl.when(i == 0)
    def _():
        stats_ref[...] = jnp.zeros(stats_ref.shape, stats_ref.dtype)
    stats_ref[0:1] += jnp.sum(a, axis=0, keepdims=True)
    stats_ref[1:2] += jnp.sum(a * a, axis=0, keepdims=True)
    stats_ref[2:3] += jnp.sum(b, axis=0, keepdims=True)
    stats_ref[3:4] += jnp.sum(b * b, axis=0, keepdims=True)


def _bn_coefs(stats_ref, row, g, bt):
    mu = stats_ref[row:row + 1] * (1.0 / N)
    var = stats_ref[row + 1:row + 2] * (1.0 / N) - mu * mu
    scale = g * lax.rsqrt(var + 1e-5)
    return scale, bt - mu * scale


def _tc_pre1(x_ref, a0_ref, a1_ref, d0_ref, d1_ref, ma_ref,
             w1l, w1r, b1, w2l, w2r, b2,
             hp_ref, st_ref, r_ref, stats_ref):
    i = pl.program_id(0)
    x = x_ref[...]
    ma = ma_ref[...]
    t = jnp.dot(x, ma, precision=_HI)
    s = x - t
    aggx = a0_ref[...] + a1_ref[...]
    r = 1.0 / jnp.maximum(d0_ref[...] + d1_ref[...], 1.0)
    r_ref[...] = r
    aggt = jnp.dot(aggx, ma, precision=_HI)
    h1p = jnp.dot((aggx - aggt) * r, w1l[...], precision=_HI) + b1[...] \
        + jnp.dot(s, w1r[...], precision=_HI)
    h2p = jnp.dot(aggt * r, w2l[...], precision=_HI) + b2[...] \
        + jnp.dot(t, w2r[...], precision=_HI)
    hp_ref[0] = h1p
    hp_ref[1] = h2p
    st_ref[0] = s
    st_ref[1] = t
    _stats_accum(stats_ref, i, h1p, h2p)


def _tc_bnrelu(hp_ref, stats_ref, g1, bt1, g2, bt2, h_ref):
    sc1, sh1 = _bn_coefs(stats_ref, 0, g1[...], bt1[...])
    sc2, sh2 = _bn_coefs(stats_ref, 2, g2[...], bt2[...])
    h_ref[0] = jax.nn.relu(hp_ref[0] * sc1 + sh1)
    h_ref[1] = jax.nn.relu(hp_ref[1] * sc2 + sh2)


def _tc_pre2(h_ref, a0_ref, a1_ref, r_ref, w1l, w1r, b1, w2l, w2r, b2,
             op_ref, stats_ref):
    i = pl.program_id(0)
    r = r_ref[...]
    o1p = jnp.dot(a0_ref[...] * r, w1l[...], precision=_HI) + b1[...] \
        + jnp.dot(h_ref[0], w1r[...], precision=_HI)
    o2p = jnp.dot(a1_ref[...] * r, w2l[...], precision=_HI) + b2[...] \
        + jnp.dot(h_ref[1], w2r[...], precision=_HI)
    op_ref[0] = o1p
    op_ref[1] = o2p
    _stats_accum(stats_ref, i, o1p, o2p)


def _tc_final(op_ref, st_ref, stats_ref, g1, bt1, g2, bt2, out_ref):
    sc1, sh1 = _bn_coefs(stats_ref, 0, g1[...], bt1[...])
    sc2, sh2 = _bn_coefs(stats_ref, 2, g2[...], bt2[...])
    o1 = jax.nn.relu(op_ref[0] * sc1 + sh1 + st_ref[0])
    o2 = jax.nn.relu(op_ref[1] * sc2 + sh2 + st_ref[1])
    out_ref[...] = o1 + o2


def kernel(x, edge_index, W11l, W11r, b11, g11, bt11, W12l, W12r, b12, g12,
           bt12, W21l, W21r, b21, g21, bt21, W22l, W22r, b22, g22, bt22):
    src = edge_index[0]
    dst = edge_index[1]
    srcs1, dsts1 = _pad_edges(src, dst, EP1, 2)
    src_p, dst_p = _pad_edges(src, dst, EP2, 1)
    srcs2 = jnp.concatenate([src_p, src_p + N])
    dsts2 = jnp.concatenate([dst_p, dst_p])
    zeros = jnp.zeros((N_PAD, C), jnp.float32)
    zeros1 = jnp.zeros((N_PAD,), jnp.float32)

    mat = _fix_spec((C, C))
    vec = _fix_spec((1, C))
    stats_spec = _fix_spec((8, C))
    row = _row_spec((BR, C))
    row1 = _row_spec((BR, 1))
    stk = _stk_spec((2, BR, C))

    aggx, degp = _sc_agg1(x, srcs1, dsts1, zeros, zeros1)

    hp, st, r, stats1 = pl.pallas_call(
        _tc_pre1,
        grid=(NB,),
        in_specs=[row, row, row, row1, row1, mat,
                  mat, mat, vec, mat, mat, vec],
        out_specs=[stk, stk, row1, stats_spec],
        out_shape=[jax.ShapeDtypeStruct((2, N, C), jnp.float32),
                   jax.ShapeDtypeStruct((2, N, C), jnp.float32),
                   jax.ShapeDtypeStruct((N, 1), jnp.float32),
                   jax.ShapeDtypeStruct((8, C), jnp.float32)],
        name="tc_pre1",
    )(x, aggx[0, :N], aggx[1, :N], degp[0, :N, None], degp[1, :N, None], _MA,
      W11l.T, W11r.T, b11[None, :], W21l.T, W21r.T, b21[None, :])

    h = pl.pallas_call(
        _tc_bnrelu,
        grid=(NB,),
        in_specs=[stk, stats_spec, vec, vec, vec, vec],
        out_specs=stk,
        out_shape=jax.ShapeDtypeStruct((2, N, C), jnp.float32),
        name="tc_bnrelu",
    )(hp, stats1, g11[None, :], bt11[None, :], g21[None, :], bt21[None, :])

    (agg2,) = _sc_agg2(h.reshape(2 * N, C), srcs2, dsts2, zeros)

    op, stats2 = pl.pallas_call(
        _tc_pre2,
        grid=(NB,),
        in_specs=[stk, row, row, row1, mat, mat, vec, mat, mat, vec],
        out_specs=[stk, stats_spec],
        out_shape=[jax.ShapeDtypeStruct((2, N, C), jnp.float32),
                   jax.ShapeDtypeStruct((8, C), jnp.float32)],
        name="tc_pre2",
    )(h, agg2[0, :N], agg2[1, :N], r,
      W12l.T, W12r.T, b12[None, :], W22l.T, W22r.T, b22[None, :])

    return pl.pallas_call(
        _tc_final,
        grid=(NB,),
        in_specs=[stk, stk, stats_spec, vec, vec, vec, vec],
        out_specs=row,
        out_shape=jax.ShapeDtypeStruct((N, C), jnp.float32),
        name="tc_final",
    )(op, st, stats2, g12[None, :], bt12[None, :], g22[None, :], bt22[None, :])
